# baseline (speedup 1.0000x reference)
import jax
import jax.numpy as jnp
from jax import lax
from jax.experimental import pallas as pl
from jax.experimental.pallas import tpu as pltpu

OUT_D = 1000
B, L = 4096, 20
ROWS = B * L
BR = 1024
NBLK = ROWS // BR
NBUF = 6
NXBUF = 2


def _body(x_hbm, o_ref, scratch, xbuf, sems, xsems):
    def start_x(i):
        pltpu.make_async_copy(
            x_hbm.at[pl.ds(i * BR, BR), :],
            xbuf.at[lax.rem(i, NXBUF)],
            xsems.at[lax.rem(i, NXBUF)],
        ).start()

    start_x(0)
    start_x(1)

    def step(i, carry):
        slot = lax.rem(i, NBUF)
        xslot = lax.rem(i, NXBUF)

        @pl.when(i >= NBUF)
        def _wait_prev_out():
            pltpu.make_async_copy(
                scratch.at[slot],
                o_ref.at[pl.ds((i - NBUF) * BR, BR)],
                sems.at[slot],
            ).wait()

        pltpu.make_async_copy(
            x_hbm.at[pl.ds(i * BR, BR), :],
            xbuf.at[xslot],
            xsems.at[xslot],
        ).wait()
        xb = xbuf[xslot]
        iota = lax.broadcasted_iota(jnp.int32, (BR, OUT_D), 1)
        scratch[slot] = (iota == xb).astype(jnp.float32)
        pltpu.make_async_copy(
            scratch.at[slot],
            o_ref.at[pl.ds(i * BR, BR)],
            sems.at[slot],
        ).start()

        @pl.when(i + NXBUF < NBLK)
        def _prefetch_x():
            start_x(i + NXBUF)

        return carry

    lax.fori_loop(0, NBLK, step, 0)

    def drain(i, carry):
        slot = lax.rem(i, NBUF)
        pltpu.make_async_copy(
            scratch.at[slot],
            o_ref.at[pl.ds(i * BR, BR)],
            sems.at[slot],
        ).wait()
        return carry

    lax.fori_loop(NBLK - NBUF, NBLK, drain, 0)


def kernel(x):
    xf = x.reshape(ROWS, 1)
    out = pl.pallas_call(
        _body,
        in_specs=[pl.BlockSpec(memory_space=pltpu.MemorySpace.HBM)],
        out_specs=pl.BlockSpec(memory_space=pltpu.MemorySpace.HBM),
        out_shape=jax.ShapeDtypeStruct((ROWS, OUT_D), jnp.float32),
        scratch_shapes=[
            pltpu.VMEM((NBUF, BR, OUT_D), jnp.float32),
            pltpu.VMEM((NXBUF, BR, 1), jnp.int32),
            pltpu.SemaphoreType.DMA((NBUF,)),
            pltpu.SemaphoreType.DMA((NXBUF,)),
        ],
    )(xf)
    return out.reshape(B, L, OUT_D)
